# baseline (speedup 1.0000x reference)
import functools

import jax
import jax.numpy as jnp
from jax import lax
from jax.experimental import pallas as pl
from jax.experimental.pallas import tpu as pltpu
from jax.experimental.pallas import tpu_sc as plsc

N_NODES = 10000
NP = 10240
D = 128
DH = D // 2
NM = 320000
NC = 2
NS = 16
CH = NM // NS
R = 128
NBF = 156
RREM = CH - NBF * R
NRING = 4
LOOK = 2
PPC = NM // (NC * NS)
PC = 2000
NPC = PPC // PC


def _iota16():
    return lax.iota(jnp.int32, 16)


def _stage_a_body(msg_hbm, nid_hbm, ts_hbm, sums_out, counts_out, tsp_out,
                  accA, accB, bufsA, bufsB, idxs, bufrA, bufrB, idxr,
                  nbuf, tsb, cnt_loc, ts_loc, sin, semA, semB, semp):
    c = lax.axis_index("c")
    s = lax.axis_index("s")
    zf = jnp.zeros((16,), jnp.float32)
    ones = jnp.ones((16,), jnp.float32)
    base = s * CH

    def issue_in(b, k):
        pltpu.async_copy(
            msg_hbm.at[pl.ds(base + b * R, R), pl.ds(c * DH, 32)],
            bufsA[k], sin[k])
        pltpu.async_copy(
            msg_hbm.at[pl.ds(base + b * R, R), pl.ds(c * DH + 32, 32)],
            bufsB[k], sin[k])
        pltpu.async_copy(nid_hbm.at[pl.ds(base + b * R, R)], idxs[k],
                         sin[k])

    issue_in(0, 0)
    issue_in(1, 1)

    def zrow(i, _):
        for q in range(2):
            bufsA[3][i, pl.ds(16 * q, 16)] = zf
            bufsB[3][i, pl.ds(16 * q, 16)] = zf
        return 0
    lax.fori_loop(0, R, zrow, 0)
    for k in range(5):
        pltpu.sync_copy(bufsA[3], accA.at[pl.ds(s * 640 + k * R, R)])
        pltpu.sync_copy(bufsB[3], accB.at[pl.ds(s * 640 + k * R, R)])
    plsc.subcore_barrier()

    def _scalar():
        def zloc(j, _):
            cnt_loc[pl.ds(16 * j, 16)] = zf
            ts_loc[pl.ds(16 * j, 16)] = zf
            return 0
        lax.fori_loop(0, NP // 16, zloc, 0)
        pbase = (c * NS + s) * PPC

        def chunk(q, _):
            st = pbase + q * PC
            nbuf[pl.ds(PC, 16)] = jnp.full((16,), N_NODES, jnp.int32)
            pltpu.sync_copy(ts_hbm.at[pl.ds(st, PC)], tsb)

            @pl.when(st + PC < NM)
            def _():
                pltpu.sync_copy(nid_hbm.at[pl.ds(st, PC + 16)],
                                nbuf.at[pl.ds(0, PC + 16)])

            @pl.when(st + PC >= NM)
            def _():
                pltpu.sync_copy(nid_hbm.at[pl.ds(st, PC)],
                                nbuf.at[pl.ds(0, PC)])

            def pp(j, _):
                v = nbuf[pl.ds(16 * j, 16)]
                nx = nbuf[pl.ds(16 * j + 1, 16)]
                tv = tsb[pl.ds(16 * j, 16)]
                plsc.addupdate_scatter(cnt_loc, [v], ones)
                plsc.store_scatter(ts_loc, [v], tv, mask=(v != nx))
                return 0
            lax.fori_loop(0, PC // 16, pp, 0)
            return 0
        lax.fori_loop(0, NPC, chunk, 0)

        pltpu.async_copy(cnt_loc, counts_out.at[c, s], semp)
        pltpu.async_copy(ts_loc, tsp_out.at[c, s], semp)
    _scalar()

    def wait_in(k):
        pltpu.make_async_copy(
            msg_hbm.at[pl.ds(base, R), pl.ds(c * DH, 32)],
            bufsA[k], sin[k]).wait()
        pltpu.make_async_copy(
            msg_hbm.at[pl.ds(base, R), pl.ds(c * DH, 32)],
            bufsB[k], sin[k]).wait()
        pltpu.make_async_copy(nid_hbm.at[pl.ds(base, R)], idxs[k],
                              sin[k]).wait()

    def issue_sc(k):
        pltpu.async_copy(bufsA[k], accA.at[idxs[k]], semA, add=True)
        pltpu.async_copy(bufsB[k], accB.at[idxs[k]], semB, add=True)

    def wait_sc(k):
        pltpu.make_async_copy(bufsA[k], accA.at[idxs[k]], semA).wait()
        pltpu.make_async_copy(bufsB[k], accB.at[idxs[k]], semB).wait()

    def ring(i, _):
        for k in range(NRING):
            b = i * NRING + k
            wait_in(k)

            @pl.when(b > 0)
            def _():
                wait_sc((k - 1) % NRING)
            issue_sc(k)

            @pl.when(b + LOOK < NBF)
            def _():
                issue_in(b + LOOK, (k + LOOK) % NRING)
        return 0
    lax.fori_loop(0, NBF // NRING, ring, 0)
    wait_sc((NBF - 1) % NRING)

    pltpu.sync_copy(
        msg_hbm.at[pl.ds(base + NBF * R, RREM), pl.ds(c * DH, 32)],
        bufrA)
    pltpu.sync_copy(
        msg_hbm.at[pl.ds(base + NBF * R, RREM), pl.ds(c * DH + 32, 32)],
        bufrB)
    pltpu.sync_copy(nid_hbm.at[pl.ds(base + NBF * R, RREM)], idxr)
    pltpu.sync_copy(bufrA, accA.at[idxr], add=True)
    pltpu.sync_copy(bufrB, accB.at[idxr], add=True)

    plsc.subcore_barrier()
    pltpu.sync_copy(accA.at[pl.ds(s * 640, 640)],
                    sums_out.at[c, pl.ds(s * 640, 640), pl.ds(0, 32)])
    pltpu.sync_copy(accB.at[pl.ds(s * 640, 640)],
                    sums_out.at[c, pl.ds(s * 640, 640), pl.ds(32, 32)])
    pltpu.make_async_copy(cnt_loc, counts_out.at[c, s], semp).wait()
    pltpu.make_async_copy(ts_loc, tsp_out.at[c, s], semp).wait()


def _stage_b_body(sums_p, counts_p, tsp_p, mean_out, tso_out, nids_out,
                  cmg_sh, bufA, bufB, obuf, cAB, tAB, tob, m16b, o640,
                  merged, ulist, sin, sout):
    c = lax.axis_index("c")
    s = lax.axis_index("s")

    @pl.when(c == 0)
    def _mean():
        def issue(k, sl):
            nb = s * 640 + k * 64
            pltpu.async_copy(sums_p.at[0, pl.ds(nb, 64)], bufA[sl], sin[sl])
            pltpu.async_copy(sums_p.at[1, pl.ds(nb, 64)], bufB[sl], sin[sl])
            pltpu.async_copy(counts_p.at[:, :, pl.ds(nb, 64)], cAB[sl],
                             sin[sl])
            pltpu.async_copy(tsp_p.at[:, :, pl.ds(nb, 64)], tAB[sl],
                             sin[sl])

        def wait_issue(sl):
            pltpu.make_async_copy(sums_p.at[0, pl.ds(0, 64)], bufA[sl],
                                  sin[sl]).wait()
            pltpu.make_async_copy(sums_p.at[1, pl.ds(0, 64)], bufB[sl],
                                  sin[sl]).wait()
            pltpu.make_async_copy(counts_p.at[:, :, pl.ds(0, 64)],
                                  cAB[sl], sin[sl]).wait()
            pltpu.make_async_copy(tsp_p.at[:, :, pl.ds(0, 64)],
                                  tAB[sl], sin[sl]).wait()

        def wait_out(k, sl):
            nb = s * 640 + k * 64
            pltpu.make_async_copy(obuf[sl], mean_out.at[pl.ds(nb, 64)],
                                  sout[sl]).wait()
            pltpu.make_async_copy(tob[sl], tso_out.at[pl.ds(nb, 64)],
                                  sout[sl]).wait()

        issue(0, 0)

        def chunk(i, _):
            for sl in range(2):
                k = 2 * i + sl
                nb = s * 640 + k * 64

                @pl.when(k + 1 < 10)
                def _():
                    issue(k + 1, 1 - sl)
                wait_issue(sl)

                @pl.when(k >= 2)
                def _():
                    wait_out(k - 2, sl)

                def mrg(r, _):
                    cnt = cAB[sl][0, 0, pl.ds(16 * r, 16)]
                    tsv = tAB[sl][0, 0, pl.ds(16 * r, 16)]
                    for a in range(NC):
                        for t in range(NS):
                            if a == 0 and t == 0:
                                continue
                            cnt = cnt + cAB[sl][a, t, pl.ds(16 * r, 16)]
                            tsv = tsv + tAB[sl][a, t, pl.ds(16 * r, 16)]
                    o640[pl.ds(16 * r, 16)] = cnt
                    tob[sl][pl.ds(16 * r, 16)] = tsv
                    return 0
                lax.fori_loop(0, 4, mrg, 0)

                def row(rw, _):
                    r = lax.shift_right_logical(rw, 4)
                    j = lax.bitwise_and(rw, 15)
                    cnt = o640[pl.ds(16 * r, 16)]
                    den = jnp.maximum(cnt, 1.0)
                    rec = 1.0 / den
                    sp = lax.gather(
                        rec, jnp.full((16, 1), j, jnp.int32),
                        dimension_numbers=lax.GatherDimensionNumbers(
                            offset_dims=(), collapsed_slice_dims=(0,),
                            start_index_map=(0,)),
                        slice_sizes=(1,),
                        mode=lax.GatherScatterMode.PROMISE_IN_BOUNDS)
                    for q in range(DH // 16):
                        obuf[sl][rw, pl.ds(16 * q, 16)] = (
                            bufA[sl][rw, pl.ds(16 * q, 16)] * sp)
                        obuf[sl][rw, pl.ds(DH + 16 * q, 16)] = (
                            bufB[sl][rw, pl.ds(16 * q, 16)] * sp)
                    return 0
                lax.fori_loop(0, 64, row, 0)

                pltpu.async_copy(obuf[sl], mean_out.at[pl.ds(nb, 64)],
                                 sout[sl])
                pltpu.async_copy(tob[sl], tso_out.at[pl.ds(nb, 64)],
                                 sout[sl])
            return 0
        lax.fori_loop(0, 5, chunk, 0)
        wait_out(8, 0)
        wait_out(9, 1)

    @pl.when(c == 1)
    def _compact():
        pltpu.sync_copy(counts_p.at[:, :, pl.ds(s * 640, 640)], m16b)

        def mrow(j, _):
            acc = m16b[0, 0, pl.ds(16 * j, 16)]
            for a in range(NC):
                for t in range(NS):
                    if a == 0 and t == 0:
                        continue
                    acc = acc + m16b[a, t, pl.ds(16 * j, 16)]
            o640[pl.ds(16 * j, 16)] = acc
            return 0
        lax.fori_loop(0, 40, mrow, 0)
        pltpu.sync_copy(o640, cmg_sh.at[pl.ds(s * 640, 640)])
        plsc.subcore_barrier()

        @pl.when(s == 0)
        def _seq():
            pltpu.sync_copy(cmg_sh, merged)
            neg1 = jnp.full((16,), -1, jnp.int32)

            def pre(j, _):
                ulist[pl.ds(16 * j, 16)] = neg1
                return 0
            lax.fori_loop(0, (N_NODES + 16) // 16, pre, 0)

            def step(j, off):
                cnt = merged[pl.ds(16 * j, 16)]
                m = cnt > 0.0
                nidv = _iota16() + 16 * j
                plsc.store_compressed(ulist.at[pl.ds(off, 16)], nidv,
                                      mask=m)
                pc = plsc.all_reduce_population_count(m)
                return off + jnp.max(pc)
            lax.fori_loop(0, N_NODES // 16, step, jnp.int32(0))
            pltpu.sync_copy(ulist.at[pl.ds(0, N_NODES)], nids_out)


def _mesh():
    return plsc.VectorSubcoreMesh(core_axis_name="c", subcore_axis_name="s",
                                  num_cores=NC, num_subcores=NS)


@functools.cache
def _build_stage_a():
    return pl.kernel(
        _stage_a_body,
        out_type=(
            jax.ShapeDtypeStruct((NC, NP, DH), jnp.float32),
            jax.ShapeDtypeStruct((NC, NS, NP), jnp.float32),
            jax.ShapeDtypeStruct((NC, NS, NP), jnp.float32),
        ),
        mesh=_mesh(),
        scratch_types=[
            pltpu.VMEM_SHARED((NP, DH // 2), jnp.float32),
            pltpu.VMEM_SHARED((NP, DH // 2), jnp.float32),
            [pltpu.VMEM((R, DH // 2), jnp.float32)] * NRING,
            [pltpu.VMEM((R, DH // 2), jnp.float32)] * NRING,
            [pltpu.VMEM((R,), jnp.int32)] * NRING,
            pltpu.VMEM((RREM, DH // 2), jnp.float32),
            pltpu.VMEM((RREM, DH // 2), jnp.float32),
            pltpu.VMEM((RREM,), jnp.int32),
            pltpu.VMEM((PC + 16,), jnp.int32),
            pltpu.VMEM((PC,), jnp.float32),
            pltpu.VMEM((NP,), jnp.float32),
            pltpu.VMEM((NP,), jnp.float32),
            [pltpu.SemaphoreType.DMA] * NRING,
            pltpu.SemaphoreType.DMA,
            pltpu.SemaphoreType.DMA,
            pltpu.SemaphoreType.DMA,
        ],
        compiler_params=pltpu.CompilerParams(needs_layout_passes=False,
                                             use_tc_tiling_on_sc=False),
    )


@functools.cache
def _build_stage_b():
    return pl.kernel(
        _stage_b_body,
        out_type=(
            jax.ShapeDtypeStruct((NP, D), jnp.float32),
            jax.ShapeDtypeStruct((NP,), jnp.float32),
            jax.ShapeDtypeStruct((N_NODES,), jnp.int32),
        ),
        mesh=_mesh(),
        scratch_types=[
            pltpu.VMEM_SHARED((NP,), jnp.float32),
            [pltpu.VMEM((64, DH), jnp.float32)] * 2,
            [pltpu.VMEM((64, DH), jnp.float32)] * 2,
            [pltpu.VMEM((64, D), jnp.float32)] * 2,
            [pltpu.VMEM((NC, NS, 64), jnp.float32)] * 2,
            [pltpu.VMEM((NC, NS, 64), jnp.float32)] * 2,
            [pltpu.VMEM((64,), jnp.float32)] * 2,
            pltpu.VMEM((NC, NS, 640), jnp.float32),
            pltpu.VMEM((640,), jnp.float32),
            pltpu.VMEM((NP,), jnp.float32),
            pltpu.VMEM((N_NODES + 16,), jnp.int32),
            [pltpu.SemaphoreType.DMA] * 2,
            [pltpu.SemaphoreType.DMA] * 2,
        ],
        compiler_params=pltpu.CompilerParams(needs_layout_passes=False,
                                             use_tc_tiling_on_sc=False),
    )


def kernel(messages, msg_nids, timestamps):
    sums, counts, tsp = _build_stage_a()(messages, msg_nids, timestamps)
    mean, tso, nids = _build_stage_b()(sums, counts, tsp)
    return (nids, mean[:N_NODES], tso[:N_NODES])


# speedup vs baseline: 1.0535x; 1.0535x over previous
import functools

import jax
import jax.numpy as jnp
from jax import lax
from jax.experimental import pallas as pl
from jax.experimental.pallas import tpu as pltpu
from jax.experimental.pallas import tpu_sc as plsc

N_NODES = 10000
NP = 10240
D = 128
DH = D // 2
NM = 320000
NC = 2
NS = 16
CH = NM // NS
R = 128
NBF = 156
RREM = CH - NBF * R
NRING = 4
LOOK = 2
PPC = NM // (NC * NS)
PC = 2000
NPC = PPC // PC


def _iota16():
    return lax.iota(jnp.int32, 16)


def _stage_a_body(msg_hbm, nid_hbm, ts_hbm, sums_out, counts_out, tsp_out,
                  acc_sh, bufs, idxs, bufr, idxr, nbuf, tsb, cnt_loc,
                  ts_loc, sin, sout, semp):
    c = lax.axis_index("c")
    s = lax.axis_index("s")
    zf = jnp.zeros((16,), jnp.float32)
    ones = jnp.ones((16,), jnp.float32)
    base = s * CH

    def issue_in(b, k):
        pltpu.async_copy(
            msg_hbm.at[pl.ds(base + b * R, R), pl.ds(c * DH, DH)],
            bufs[k], sin[k])
        pltpu.async_copy(nid_hbm.at[pl.ds(base + b * R, R)], idxs[k],
                         sin[k])

    issue_in(0, 0)
    issue_in(1, 1)

    def zrow(i, _):
        for q in range(DH // 16):
            bufs[3][i, pl.ds(16 * q, 16)] = zf
        return 0
    lax.fori_loop(0, R, zrow, 0)
    for k in range(5):
        pltpu.sync_copy(bufs[3], acc_sh.at[pl.ds(s * 640 + k * R, R)])
    plsc.subcore_barrier()

    def _scalar():
        def zloc(j, _):
            cnt_loc[pl.ds(16 * j, 16)] = zf
            ts_loc[pl.ds(16 * j, 16)] = zf
            return 0
        lax.fori_loop(0, NP // 16, zloc, 0)
        pbase = (c * NS + s) * PPC

        def chunk(q, _):
            st = pbase + q * PC
            nbuf[pl.ds(PC, 16)] = jnp.full((16,), N_NODES, jnp.int32)
            pltpu.sync_copy(ts_hbm.at[pl.ds(st, PC)], tsb)

            @pl.when(st + PC < NM)
            def _():
                pltpu.sync_copy(nid_hbm.at[pl.ds(st, PC + 16)],
                                nbuf.at[pl.ds(0, PC + 16)])

            @pl.when(st + PC >= NM)
            def _():
                pltpu.sync_copy(nid_hbm.at[pl.ds(st, PC)],
                                nbuf.at[pl.ds(0, PC)])

            def pp(j, _):
                v = nbuf[pl.ds(16 * j, 16)]
                nx = nbuf[pl.ds(16 * j + 1, 16)]
                tv = tsb[pl.ds(16 * j, 16)]
                plsc.addupdate_scatter(cnt_loc, [v], ones)
                plsc.store_scatter(ts_loc, [v], tv, mask=(v != nx))
                return 0
            lax.fori_loop(0, PC // 16, pp, 0)
            return 0
        lax.fori_loop(0, NPC, chunk, 0)

        pltpu.async_copy(cnt_loc, counts_out.at[c, s], semp)
        pltpu.async_copy(ts_loc, tsp_out.at[c, s], semp)
    _scalar()

    def wait_in(k):
        pltpu.make_async_copy(
            msg_hbm.at[pl.ds(base, R), pl.ds(c * DH, DH)],
            bufs[k], sin[k]).wait()
        pltpu.make_async_copy(nid_hbm.at[pl.ds(base, R)], idxs[k],
                              sin[k]).wait()

    def issue_sc(k):
        pltpu.async_copy(bufs[k], acc_sh.at[idxs[k]], sout[k], add=True)

    def wait_sc(k):
        pltpu.make_async_copy(bufs[k], acc_sh.at[idxs[k]],
                              sout[k]).wait()

    def ring(i, _):
        for k in range(NRING):
            b = i * NRING + k
            wait_in(k)
            issue_sc(k)

            @pl.when(b + LOOK < NBF)
            def _():
                kk = (k + LOOK) % NRING

                @pl.when(b + LOOK >= NRING)
                def _():
                    wait_sc(kk)
                issue_in(b + LOOK, kk)
        return 0
    lax.fori_loop(0, NBF // NRING, ring, 0)
    for k in range(NRING):
        wait_sc(k)

    pltpu.sync_copy(
        msg_hbm.at[pl.ds(base + NBF * R, RREM), pl.ds(c * DH, DH)], bufr)
    pltpu.sync_copy(nid_hbm.at[pl.ds(base + NBF * R, RREM)], idxr)
    pltpu.sync_copy(bufr, acc_sh.at[idxr], add=True)

    plsc.subcore_barrier()
    pltpu.sync_copy(acc_sh.at[pl.ds(s * 640, 640)],
                    sums_out.at[c, pl.ds(s * 640, 640)])
    pltpu.make_async_copy(cnt_loc, counts_out.at[c, s], semp).wait()
    pltpu.make_async_copy(ts_loc, tsp_out.at[c, s], semp).wait()


def _stage_b_body(sums_p, counts_p, tsp_p, mean_out, tso_out, nids_out,
                  cmg_sh, bufA, bufB, obuf, cAB, tAB, tob, m16b, o640,
                  merged, ulist, sin, sout):
    c = lax.axis_index("c")
    s = lax.axis_index("s")

    @pl.when(c == 0)
    def _mean():
        def issue(k, sl):
            nb = s * 640 + k * 64
            pltpu.async_copy(sums_p.at[0, pl.ds(nb, 64)], bufA[sl], sin[sl])
            pltpu.async_copy(sums_p.at[1, pl.ds(nb, 64)], bufB[sl], sin[sl])
            pltpu.async_copy(counts_p.at[:, :, pl.ds(nb, 64)], cAB[sl],
                             sin[sl])
            pltpu.async_copy(tsp_p.at[:, :, pl.ds(nb, 64)], tAB[sl],
                             sin[sl])

        def wait_issue(sl):
            pltpu.make_async_copy(sums_p.at[0, pl.ds(0, 64)], bufA[sl],
                                  sin[sl]).wait()
            pltpu.make_async_copy(sums_p.at[1, pl.ds(0, 64)], bufB[sl],
                                  sin[sl]).wait()
            pltpu.make_async_copy(counts_p.at[:, :, pl.ds(0, 64)],
                                  cAB[sl], sin[sl]).wait()
            pltpu.make_async_copy(tsp_p.at[:, :, pl.ds(0, 64)],
                                  tAB[sl], sin[sl]).wait()

        def wait_out(k, sl):
            nb = s * 640 + k * 64
            pltpu.make_async_copy(obuf[sl], mean_out.at[pl.ds(nb, 64)],
                                  sout[sl]).wait()
            pltpu.make_async_copy(tob[sl], tso_out.at[pl.ds(nb, 64)],
                                  sout[sl]).wait()

        issue(0, 0)

        def chunk(i, _):
            for sl in range(2):
                k = 2 * i + sl
                nb = s * 640 + k * 64

                @pl.when(k + 1 < 10)
                def _():
                    issue(k + 1, 1 - sl)
                wait_issue(sl)

                @pl.when(k >= 2)
                def _():
                    wait_out(k - 2, sl)

                def mrg(r, _):
                    cnt = cAB[sl][0, 0, pl.ds(16 * r, 16)]
                    tsv = tAB[sl][0, 0, pl.ds(16 * r, 16)]
                    for a in range(NC):
                        for t in range(NS):
                            if a == 0 and t == 0:
                                continue
                            cnt = cnt + cAB[sl][a, t, pl.ds(16 * r, 16)]
                            tsv = tsv + tAB[sl][a, t, pl.ds(16 * r, 16)]
                    o640[pl.ds(16 * r, 16)] = cnt
                    tob[sl][pl.ds(16 * r, 16)] = tsv
                    return 0
                lax.fori_loop(0, 4, mrg, 0)

                def row(rw, _):
                    r = lax.shift_right_logical(rw, 4)
                    j = lax.bitwise_and(rw, 15)
                    cnt = o640[pl.ds(16 * r, 16)]
                    den = jnp.maximum(cnt, 1.0)
                    rec = 1.0 / den
                    sp = lax.gather(
                        rec, jnp.full((16, 1), j, jnp.int32),
                        dimension_numbers=lax.GatherDimensionNumbers(
                            offset_dims=(), collapsed_slice_dims=(0,),
                            start_index_map=(0,)),
                        slice_sizes=(1,),
                        mode=lax.GatherScatterMode.PROMISE_IN_BOUNDS)
                    for q in range(DH // 16):
                        obuf[sl][rw, pl.ds(16 * q, 16)] = (
                            bufA[sl][rw, pl.ds(16 * q, 16)] * sp)
                        obuf[sl][rw, pl.ds(DH + 16 * q, 16)] = (
                            bufB[sl][rw, pl.ds(16 * q, 16)] * sp)
                    return 0
                lax.fori_loop(0, 64, row, 0)

                pltpu.async_copy(obuf[sl], mean_out.at[pl.ds(nb, 64)],
                                 sout[sl])
                pltpu.async_copy(tob[sl], tso_out.at[pl.ds(nb, 64)],
                                 sout[sl])
            return 0
        lax.fori_loop(0, 5, chunk, 0)
        wait_out(8, 0)
        wait_out(9, 1)

    @pl.when(c == 1)
    def _compact():
        pltpu.sync_copy(counts_p.at[:, :, pl.ds(s * 640, 640)], m16b)

        def mrow(j, _):
            acc = m16b[0, 0, pl.ds(16 * j, 16)]
            for a in range(NC):
                for t in range(NS):
                    if a == 0 and t == 0:
                        continue
                    acc = acc + m16b[a, t, pl.ds(16 * j, 16)]
            o640[pl.ds(16 * j, 16)] = acc
            return 0
        lax.fori_loop(0, 40, mrow, 0)
        pltpu.sync_copy(o640, cmg_sh.at[pl.ds(s * 640, 640)])
        plsc.subcore_barrier()

        @pl.when(s == 0)
        def _seq():
            pltpu.sync_copy(cmg_sh, merged)
            neg1 = jnp.full((16,), -1, jnp.int32)

            def pre(j, _):
                ulist[pl.ds(16 * j, 16)] = neg1
                return 0
            lax.fori_loop(0, (N_NODES + 16) // 16, pre, 0)

            def step(j, off):
                cnt = merged[pl.ds(16 * j, 16)]
                m = cnt > 0.0
                nidv = _iota16() + 16 * j
                plsc.store_compressed(ulist.at[pl.ds(off, 16)], nidv,
                                      mask=m)
                pc = plsc.all_reduce_population_count(m)
                return off + jnp.max(pc)
            lax.fori_loop(0, N_NODES // 16, step, jnp.int32(0))
            pltpu.sync_copy(ulist.at[pl.ds(0, N_NODES)], nids_out)


def _mesh():
    return plsc.VectorSubcoreMesh(core_axis_name="c", subcore_axis_name="s",
                                  num_cores=NC, num_subcores=NS)


@functools.cache
def _build_stage_a():
    return pl.kernel(
        _stage_a_body,
        out_type=(
            jax.ShapeDtypeStruct((NC, NP, DH), jnp.float32),
            jax.ShapeDtypeStruct((NC, NS, NP), jnp.float32),
            jax.ShapeDtypeStruct((NC, NS, NP), jnp.float32),
        ),
        mesh=_mesh(),
        scratch_types=[
            pltpu.VMEM_SHARED((NP, DH), jnp.float32),
            [pltpu.VMEM((R, DH), jnp.float32)] * NRING,
            [pltpu.VMEM((R,), jnp.int32)] * NRING,
            pltpu.VMEM((RREM, DH), jnp.float32),
            pltpu.VMEM((RREM,), jnp.int32),
            pltpu.VMEM((PC + 16,), jnp.int32),
            pltpu.VMEM((PC,), jnp.float32),
            pltpu.VMEM((NP,), jnp.float32),
            pltpu.VMEM((NP,), jnp.float32),
            [pltpu.SemaphoreType.DMA] * NRING,
            [pltpu.SemaphoreType.DMA] * NRING,
            pltpu.SemaphoreType.DMA,
        ],
        compiler_params=pltpu.CompilerParams(needs_layout_passes=False,
                                             use_tc_tiling_on_sc=False),
    )


@functools.cache
def _build_stage_b():
    return pl.kernel(
        _stage_b_body,
        out_type=(
            jax.ShapeDtypeStruct((NP, D), jnp.float32),
            jax.ShapeDtypeStruct((NP,), jnp.float32),
            jax.ShapeDtypeStruct((N_NODES,), jnp.int32),
        ),
        mesh=_mesh(),
        scratch_types=[
            pltpu.VMEM_SHARED((NP,), jnp.float32),
            [pltpu.VMEM((64, DH), jnp.float32)] * 2,
            [pltpu.VMEM((64, DH), jnp.float32)] * 2,
            [pltpu.VMEM((64, D), jnp.float32)] * 2,
            [pltpu.VMEM((NC, NS, 64), jnp.float32)] * 2,
            [pltpu.VMEM((NC, NS, 64), jnp.float32)] * 2,
            [pltpu.VMEM((64,), jnp.float32)] * 2,
            pltpu.VMEM((NC, NS, 640), jnp.float32),
            pltpu.VMEM((640,), jnp.float32),
            pltpu.VMEM((NP,), jnp.float32),
            pltpu.VMEM((N_NODES + 16,), jnp.int32),
            [pltpu.SemaphoreType.DMA] * 2,
            [pltpu.SemaphoreType.DMA] * 2,
        ],
        compiler_params=pltpu.CompilerParams(needs_layout_passes=False,
                                             use_tc_tiling_on_sc=False),
    )


def kernel(messages, msg_nids, timestamps):
    sums, counts, tsp = _build_stage_a()(messages, msg_nids, timestamps)
    mean, tso, nids = _build_stage_b()(sums, counts, tsp)
    return (nids, mean[:N_NODES], tso[:N_NODES])


# speedup vs baseline: 1.1134x; 1.0569x over previous
import functools

import jax
import jax.numpy as jnp
from jax import lax
from jax.experimental import pallas as pl
from jax.experimental.pallas import tpu as pltpu
from jax.experimental.pallas import tpu_sc as plsc

N_NODES = 10000
NP = 10240
D = 128
DH = D // 2
NM = 320000
NC = 2
NS = 16
CH = NM // NS
R = 128
NBF = 156
RREM = CH - NBF * R
NRING = 4
LOOK = 2
PPC = NM // (NC * NS)
PC = 2000
NPC = PPC // PC


def _iota16():
    return lax.iota(jnp.int32, 16)


def _stage_a_body(msg_hbm, nid_hbm, ts_hbm, sums_out, counts_out, tsp_out,
                  acc_sh, bufs, idxs, bufr, idxr, nbuf, tsb, cnt_loc,
                  ts_loc, sin, semp):
    c = lax.axis_index("c")
    s = lax.axis_index("s")
    zf = jnp.zeros((16,), jnp.float32)
    ones = jnp.ones((16,), jnp.float32)
    base = s * CH

    def issue_in(b, k):
        pltpu.async_copy(
            msg_hbm.at[pl.ds(base + b * R, R), pl.ds(c * DH, DH)],
            bufs[k], sin[k])
        pltpu.async_copy(nid_hbm.at[pl.ds(base + b * R, R)], idxs[k],
                         sin[k])

    issue_in(0, 0)
    issue_in(1, 1)

    def zrow(i, _):
        for q in range(DH // 16):
            bufs[3][i, pl.ds(16 * q, 16)] = zf
        return 0
    lax.fori_loop(0, R, zrow, 0)
    for k in range(5):
        pltpu.sync_copy(bufs[3], acc_sh.at[pl.ds(s * 640 + k * R, R)])
    plsc.subcore_barrier()

    def _scalar():
        def zloc(j, _):
            cnt_loc[pl.ds(16 * j, 16)] = zf
            ts_loc[pl.ds(16 * j, 16)] = zf
            return 0
        lax.fori_loop(0, NP // 16, zloc, 0)
        pbase = (c * NS + s) * PPC

        def chunk(q, _):
            st = pbase + q * PC
            nbuf[pl.ds(PC, 16)] = jnp.full((16,), N_NODES, jnp.int32)
            pltpu.sync_copy(ts_hbm.at[pl.ds(st, PC)], tsb)

            @pl.when(st + PC < NM)
            def _():
                pltpu.sync_copy(nid_hbm.at[pl.ds(st, PC + 16)],
                                nbuf.at[pl.ds(0, PC + 16)])

            @pl.when(st + PC >= NM)
            def _():
                pltpu.sync_copy(nid_hbm.at[pl.ds(st, PC)],
                                nbuf.at[pl.ds(0, PC)])

            def pp(j, _):
                v = nbuf[pl.ds(16 * j, 16)]
                nx = nbuf[pl.ds(16 * j + 1, 16)]
                tv = tsb[pl.ds(16 * j, 16)]
                plsc.addupdate_scatter(cnt_loc, [v], ones)
                plsc.store_scatter(ts_loc, [v], tv, mask=(v != nx))
                return 0
            lax.fori_loop(0, PC // 16, pp, 0)
            return 0
        lax.fori_loop(0, NPC, chunk, 0)

        pltpu.async_copy(cnt_loc, counts_out.at[c, s], semp)
        pltpu.async_copy(ts_loc, tsp_out.at[c, s], semp)
    _scalar()

    def wait_in(k):
        pltpu.make_async_copy(
            msg_hbm.at[pl.ds(base, R), pl.ds(c * DH, DH)],
            bufs[k], sin[k]).wait()
        pltpu.make_async_copy(nid_hbm.at[pl.ds(base, R)], idxs[k],
                              sin[k]).wait()

    def issue_sc(k):
        pltpu.sync_copy(bufs[k], acc_sh.at[idxs[k]], add=True)

    def ring(i, _):
        for k in range(NRING):
            b = i * NRING + k

            @pl.when(b + LOOK < NBF)
            def _():
                issue_in(b + LOOK, (k + LOOK) % NRING)
            wait_in(k)
            issue_sc(k)
        return 0
    lax.fori_loop(0, NBF // NRING, ring, 0)

    pltpu.sync_copy(
        msg_hbm.at[pl.ds(base + NBF * R, RREM), pl.ds(c * DH, DH)], bufr)
    pltpu.sync_copy(nid_hbm.at[pl.ds(base + NBF * R, RREM)], idxr)
    pltpu.sync_copy(bufr, acc_sh.at[idxr], add=True)

    plsc.subcore_barrier()
    pltpu.sync_copy(acc_sh.at[pl.ds(s * 640, 640)],
                    sums_out.at[c, pl.ds(s * 640, 640)])
    pltpu.make_async_copy(cnt_loc, counts_out.at[c, s], semp).wait()
    pltpu.make_async_copy(ts_loc, tsp_out.at[c, s], semp).wait()


def _stage_b_body(sums_p, counts_p, tsp_p, mean_out, tso_out, nids_out,
                  cmg_sh, bufA, bufB, obuf, cAB, tAB, tob, m16b, o640,
                  merged, ulist, sin, sout):
    c = lax.axis_index("c")
    s = lax.axis_index("s")

    @pl.when(c == 0)
    def _mean():
        def issue(k, sl):
            nb = s * 640 + k * 64
            pltpu.async_copy(sums_p.at[0, pl.ds(nb, 64)], bufA[sl], sin[sl])
            pltpu.async_copy(sums_p.at[1, pl.ds(nb, 64)], bufB[sl], sin[sl])
            pltpu.async_copy(counts_p.at[:, :, pl.ds(nb, 64)], cAB[sl],
                             sin[sl])
            pltpu.async_copy(tsp_p.at[:, :, pl.ds(nb, 64)], tAB[sl],
                             sin[sl])

        def wait_issue(sl):
            pltpu.make_async_copy(sums_p.at[0, pl.ds(0, 64)], bufA[sl],
                                  sin[sl]).wait()
            pltpu.make_async_copy(sums_p.at[1, pl.ds(0, 64)], bufB[sl],
                                  sin[sl]).wait()
            pltpu.make_async_copy(counts_p.at[:, :, pl.ds(0, 64)],
                                  cAB[sl], sin[sl]).wait()
            pltpu.make_async_copy(tsp_p.at[:, :, pl.ds(0, 64)],
                                  tAB[sl], sin[sl]).wait()

        def wait_out(k, sl):
            nb = s * 640 + k * 64
            pltpu.make_async_copy(obuf[sl], mean_out.at[pl.ds(nb, 64)],
                                  sout[sl]).wait()
            pltpu.make_async_copy(tob[sl], tso_out.at[pl.ds(nb, 64)],
                                  sout[sl]).wait()

        issue(0, 0)

        def chunk(i, _):
            for sl in range(2):
                k = 2 * i + sl
                nb = s * 640 + k * 64

                @pl.when(k + 1 < 10)
                def _():
                    issue(k + 1, 1 - sl)
                wait_issue(sl)

                @pl.when(k >= 2)
                def _():
                    wait_out(k - 2, sl)

                def mrg(r, _):
                    cnt = cAB[sl][0, 0, pl.ds(16 * r, 16)]
                    tsv = tAB[sl][0, 0, pl.ds(16 * r, 16)]
                    for a in range(NC):
                        for t in range(NS):
                            if a == 0 and t == 0:
                                continue
                            cnt = cnt + cAB[sl][a, t, pl.ds(16 * r, 16)]
                            tsv = tsv + tAB[sl][a, t, pl.ds(16 * r, 16)]
                    o640[pl.ds(16 * r, 16)] = cnt
                    tob[sl][pl.ds(16 * r, 16)] = tsv
                    return 0
                lax.fori_loop(0, 4, mrg, 0)

                def row(rw, _):
                    r = lax.shift_right_logical(rw, 4)
                    j = lax.bitwise_and(rw, 15)
                    cnt = o640[pl.ds(16 * r, 16)]
                    den = jnp.maximum(cnt, 1.0)
                    rec = 1.0 / den
                    sp = lax.gather(
                        rec, jnp.full((16, 1), j, jnp.int32),
                        dimension_numbers=lax.GatherDimensionNumbers(
                            offset_dims=(), collapsed_slice_dims=(0,),
                            start_index_map=(0,)),
                        slice_sizes=(1,),
                        mode=lax.GatherScatterMode.PROMISE_IN_BOUNDS)
                    for q in range(DH // 16):
                        obuf[sl][rw, pl.ds(16 * q, 16)] = (
                            bufA[sl][rw, pl.ds(16 * q, 16)] * sp)
                        obuf[sl][rw, pl.ds(DH + 16 * q, 16)] = (
                            bufB[sl][rw, pl.ds(16 * q, 16)] * sp)
                    return 0
                lax.fori_loop(0, 64, row, 0)

                pltpu.async_copy(obuf[sl], mean_out.at[pl.ds(nb, 64)],
                                 sout[sl])
                pltpu.async_copy(tob[sl], tso_out.at[pl.ds(nb, 64)],
                                 sout[sl])
            return 0
        lax.fori_loop(0, 5, chunk, 0)
        wait_out(8, 0)
        wait_out(9, 1)

    @pl.when(c == 1)
    def _compact():
        pltpu.sync_copy(counts_p.at[:, :, pl.ds(s * 640, 640)], m16b)

        def mrow(j, _):
            acc = m16b[0, 0, pl.ds(16 * j, 16)]
            for a in range(NC):
                for t in range(NS):
                    if a == 0 and t == 0:
                        continue
                    acc = acc + m16b[a, t, pl.ds(16 * j, 16)]
            o640[pl.ds(16 * j, 16)] = acc
            return 0
        lax.fori_loop(0, 40, mrow, 0)
        pltpu.sync_copy(o640, cmg_sh.at[pl.ds(s * 640, 640)])
        plsc.subcore_barrier()

        @pl.when(s == 0)
        def _seq():
            pltpu.sync_copy(cmg_sh, merged)
            neg1 = jnp.full((16,), -1, jnp.int32)

            def pre(j, _):
                ulist[pl.ds(16 * j, 16)] = neg1
                return 0
            lax.fori_loop(0, (N_NODES + 16) // 16, pre, 0)

            def step(j, off):
                cnt = merged[pl.ds(16 * j, 16)]
                m = cnt > 0.0
                nidv = _iota16() + 16 * j
                plsc.store_compressed(ulist.at[pl.ds(off, 16)], nidv,
                                      mask=m)
                pc = plsc.all_reduce_population_count(m)
                return off + jnp.max(pc)
            lax.fori_loop(0, N_NODES // 16, step, jnp.int32(0))
            pltpu.sync_copy(ulist.at[pl.ds(0, N_NODES)], nids_out)


def _mesh():
    return plsc.VectorSubcoreMesh(core_axis_name="c", subcore_axis_name="s",
                                  num_cores=NC, num_subcores=NS)


@functools.cache
def _build_stage_a():
    return pl.kernel(
        _stage_a_body,
        out_type=(
            jax.ShapeDtypeStruct((NC, NP, DH), jnp.float32),
            jax.ShapeDtypeStruct((NC, NS, NP), jnp.float32),
            jax.ShapeDtypeStruct((NC, NS, NP), jnp.float32),
        ),
        mesh=_mesh(),
        scratch_types=[
            pltpu.VMEM_SHARED((NP, DH), jnp.float32),
            [pltpu.VMEM((R, DH), jnp.float32)] * NRING,
            [pltpu.VMEM((R,), jnp.int32)] * NRING,
            pltpu.VMEM((RREM, DH), jnp.float32),
            pltpu.VMEM((RREM,), jnp.int32),
            pltpu.VMEM((PC + 16,), jnp.int32),
            pltpu.VMEM((PC,), jnp.float32),
            pltpu.VMEM((NP,), jnp.float32),
            pltpu.VMEM((NP,), jnp.float32),
            [pltpu.SemaphoreType.DMA] * NRING,
            pltpu.SemaphoreType.DMA,
        ],
        compiler_params=pltpu.CompilerParams(needs_layout_passes=False,
                                             use_tc_tiling_on_sc=False),
    )


@functools.cache
def _build_stage_b():
    return pl.kernel(
        _stage_b_body,
        out_type=(
            jax.ShapeDtypeStruct((NP, D), jnp.float32),
            jax.ShapeDtypeStruct((NP,), jnp.float32),
            jax.ShapeDtypeStruct((N_NODES,), jnp.int32),
        ),
        mesh=_mesh(),
        scratch_types=[
            pltpu.VMEM_SHARED((NP,), jnp.float32),
            [pltpu.VMEM((64, DH), jnp.float32)] * 2,
            [pltpu.VMEM((64, DH), jnp.float32)] * 2,
            [pltpu.VMEM((64, D), jnp.float32)] * 2,
            [pltpu.VMEM((NC, NS, 64), jnp.float32)] * 2,
            [pltpu.VMEM((NC, NS, 64), jnp.float32)] * 2,
            [pltpu.VMEM((64,), jnp.float32)] * 2,
            pltpu.VMEM((NC, NS, 640), jnp.float32),
            pltpu.VMEM((640,), jnp.float32),
            pltpu.VMEM((NP,), jnp.float32),
            pltpu.VMEM((N_NODES + 16,), jnp.int32),
            [pltpu.SemaphoreType.DMA] * 2,
            [pltpu.SemaphoreType.DMA] * 2,
        ],
        compiler_params=pltpu.CompilerParams(needs_layout_passes=False,
                                             use_tc_tiling_on_sc=False),
    )


def kernel(messages, msg_nids, timestamps):
    sums, counts, tsp = _build_stage_a()(messages, msg_nids, timestamps)
    mean, tso, nids = _build_stage_b()(sums, counts, tsp)
    return (nids, mean[:N_NODES], tso[:N_NODES])
